# baseline (speedup 1.0000x reference)
import jax
import jax.numpy as jnp
from jax import lax
from jax.experimental import pallas as pl
from jax.experimental.pallas import tpu as pltpu
from jax.experimental.pallas import tpu_sc as plsc

B, L, EMB = 1024, 200, 16
VOC = 1_000_000
NW = 32
SEQ_PER_W = B // NW
H0, H1 = 104, 96
SCALE = 0.5 ** 0.5

LANES = 16
NBKT = VOC // 128
BKT_PER_W = NBKT // NW
EXTRA0 = NBKT - NW * BKT_PER_W
WCH = 1024
FULL_CH = (BKT_PER_W * 128) // WCH
REM = BKT_PER_W * 128 - FULL_CH * WCH
TAIL = VOC - NBKT * 128


def _transpose_loop(in_v, stage_v, width):
    iota = lax.iota(jnp.int32, LANES)
    st_idx = [jj * 16 + iota for jj in range(8)]

    @plsc.parallel_loop(0, width // 8, 1, unroll=8)
    def tok8(j8):
        ld_base = jnp.full((LANES,), j8 * 8, jnp.int32)
        row_splat = jnp.full((LANES,), j8, jnp.int32)
        for jj in range(8):
            row = plsc.load_gather(in_v, [iota, ld_base + jj])
            plsc.store_scatter(stage_v, [row_splat, st_idx[jj]], row)


def _transpose_chunk(wt_hbm, scr_hbm, in_v, stage_v, sem, lane0, width):
    lane0 = pl.multiple_of(lane0, 128)
    pltpu.sync_copy(wt_hbm.at[:, pl.ds(lane0, width)], in_v.at[:, pl.ds(0, width)])
    _transpose_loop(in_v, stage_v, width)
    pltpu.make_async_copy(
        stage_v.at[pl.ds(0, width // 8)],
        scr_hbm.at[pl.ds(pl.multiple_of(lane0 // 8, 16), width // 8)],
        sem,
    ).start()
    pltpu.make_async_copy(
        stage_v.at[pl.ds(0, width // 8)],
        scr_hbm.at[pl.ds(pl.multiple_of(lane0 // 8, 16), width // 8)],
        sem,
    ).wait()


def _tbody(wt_hbm, tail_hbm, scr_hbm, in_v, stage_v, tail_v, gsem, osem):
    nc = 2
    wid = lax.axis_index("s") * nc + lax.axis_index("c")
    base = wid * (BKT_PER_W * 128)

    def fire_in(c):
        lane0 = pl.multiple_of(base + c * WCH, 128)
        pltpu.make_async_copy(
            wt_hbm.at[:, pl.ds(lane0, WCH)],
            in_v.at[c % 2, :, pl.ds(0, WCH)], gsem).start()

    def fire_out(c):
        lane0 = pl.multiple_of(base + c * WCH, 128)
        pltpu.make_async_copy(
            stage_v.at[c % 2],
            scr_hbm.at[pl.ds(pl.multiple_of(lane0 // 8, 16), WCH // 8)],
            osem,
        ).start()

    def wait_in():
        pltpu.make_async_copy(
            wt_hbm.at[:, pl.ds(0, WCH)],
            in_v.at[0, :, pl.ds(0, WCH)], gsem).wait()

    def wait_out():
        pltpu.make_async_copy(
            stage_v.at[0], scr_hbm.at[pl.ds(0, WCH // 8)], osem).wait()

    fire_in(0)

    def chunk(c, carry):
        @pl.when(c + 1 < FULL_CH)
        def _():
            fire_in(c + 1)

        wait_in()

        @pl.when(c >= 2)
        def _():
            wait_out()

        _transpose_loop(in_v.at[c % 2], stage_v.at[c % 2], WCH)
        fire_out(c)
        return carry

    lax.fori_loop(0, FULL_CH, chunk, 0)
    wait_out()
    wait_out()
    _transpose_chunk(wt_hbm, scr_hbm, in_v.at[0], stage_v.at[0], gsem,
                     base + FULL_CH * WCH, REM)

    @pl.when(wid == 0)
    def _extra():
        _transpose_chunk(wt_hbm, scr_hbm, in_v.at[0], stage_v.at[0], gsem,
                         NW * BKT_PER_W * 128, EXTRA0 * 128)
        pltpu.sync_copy(tail_hbm, tail_v)
        iota = lax.iota(jnp.int32, LANES)

        def tok(j, carry):
            row = tail_v[j]
            plsc.store_scatter(
                stage_v.at[0],
                [jnp.full((LANES,), j // 8, jnp.int32), (j % 8) * 16 + iota],
                row,
            )
            return carry

        lax.fori_loop(0, TAIL, tok, 0, unroll=8)
        pltpu.sync_copy(
            stage_v.at[0, pl.ds(0, TAIL * 16 // 128)],
            scr_hbm.at[pl.ds(NBKT * 128 * 16 // 128, TAIL * 16 // 128)],
        )


def _gbody(tok_hbm, word_hbm, pos_hbm, out_hbm, idx_v, rows_v, stg_v, pos_v, sem):
    nc = 2
    wid = lax.axis_index("s") * nc + lax.axis_index("c")
    b0 = wid * SEQ_PER_W

    pltpu.sync_copy(tok_hbm.at[pl.ds(b0, SEQ_PER_W)], idx_v)
    pltpu.sync_copy(pos_hbm, pos_v)

    c = jnp.full((EMB,), SCALE, dtype=jnp.float32)

    def scale_pos(l, carry):
        pos_v[l] = pos_v[l] * c
        return carry

    lax.fori_loop(0, L, scale_pos, 0, unroll=4)

    iota16 = lax.iota(jnp.int32, LANES)

    for h, (l0, hh) in enumerate(((0, H0), (H0, H1))):
        def fire(s, carry):
            pltpu.make_async_copy(
                word_hbm.at[idx_v.at[s, pl.ds(l0, hh)]],
                rows_v.at[s, pl.ds(0, hh)], sem).start()
            return carry

        lax.fori_loop(0, SEQ_PER_W, fire, 0)

        def drain(s, carry):
            pltpu.make_async_copy(
                word_hbm.at[idx_v.at[0, pl.ds(l0, hh)]],
                rows_v.at[0, pl.ds(0, hh)], sem).wait()
            return carry

        lax.fori_loop(0, SEQ_PER_W, drain, 0)

        def seq(s, carry):
            @plsc.parallel_loop(0, hh, 1, unroll=4)
            def tok(i):
                val = rows_v[s, i] * c + pos_v[l0 + i]
                plsc.store_scatter(
                    stg_v,
                    [jnp.full((LANES,), i, jnp.int32), iota16,
                     jnp.full((LANES,), s, jnp.int32)],
                    val)

            return carry

        lax.fori_loop(0, SEQ_PER_W, seq, 0)

        pltpu.sync_copy(
            stg_v.at[pl.ds(0, hh), :, pl.ds(0, SEQ_PER_W)],
            out_hbm.at[pl.ds(l0, hh), :, pl.ds(b0, SEQ_PER_W)])


@jax.jit
def _embed(tok_ids, word_table, pos_table):
    mesh = plsc.VectorSubcoreMesh(core_axis_name="c", subcore_axis_name="s")

    wt_t = word_table.T
    wt_tail = lax.slice(word_table, (NBKT * 128, 0), (VOC, EMB))

    trans = pl.kernel(
        _tbody,
        out_type=jax.ShapeDtypeStruct((VOC * EMB // 128, 128), jnp.float32),
        mesh=mesh,
        scratch_types=[
            pltpu.VMEM((2, LANES, WCH + 1), jnp.float32),
            pltpu.VMEM((2, WCH // 8, 128), jnp.float32),
            pltpu.VMEM((TAIL, EMB), jnp.float32),
            pltpu.SemaphoreType.DMA,
            pltpu.SemaphoreType.DMA,
        ],
        compiler_params=pltpu.CompilerParams(
            use_tc_tiling_on_sc=True, needs_layout_passes=False),
    )
    scratch = trans(wt_t, wt_tail)
    wt_lin = scratch.reshape(VOC, EMB)

    gather = pl.kernel(
        _gbody,
        out_type=jax.ShapeDtypeStruct((L, EMB, B), jnp.float32),
        mesh=mesh,
        scratch_types=[
            pltpu.VMEM((SEQ_PER_W, L), jnp.int32),
            pltpu.VMEM((SEQ_PER_W, H0, EMB), jnp.float32),
            pltpu.VMEM((H0, EMB, 33), jnp.float32),
            pltpu.VMEM((L, EMB), jnp.float32),
            pltpu.SemaphoreType.DMA,
        ],
        compiler_params=pltpu.CompilerParams(
            use_tc_tiling_on_sc=False, needs_layout_passes=False),
    )
    out_t = gather(tok_ids, wt_lin, pos_table)
    return jnp.transpose(out_t, (2, 0, 1))


def kernel(tok_ids, word_table, pos_table):
    return _embed(tok_ids, word_table, pos_table)


# speedup vs baseline: 1.0188x; 1.0188x over previous
import jax
import jax.numpy as jnp
from jax import lax
from jax.experimental import pallas as pl
from jax.experimental.pallas import tpu as pltpu
from jax.experimental.pallas import tpu_sc as plsc

B, L, EMB = 1024, 200, 16
VOC = 1_000_000
NW = 32
SEQ_PER_W = B // NW
H0, H1 = 104, 96
SCALE = 0.5 ** 0.5

LANES = 16
NBKT = VOC // 128
BKT_PER_W = NBKT // NW
EXTRA0 = NBKT - NW * BKT_PER_W
WCH = 1536
FULL_CH = (BKT_PER_W * 128) // WCH
REM = BKT_PER_W * 128 - FULL_CH * WCH
TAIL = VOC - NBKT * 128


def _transpose_loop(in_v, stage_v, width):
    iota = lax.iota(jnp.int32, LANES)
    st_idx = [jj * 16 + iota for jj in range(8)]

    @plsc.parallel_loop(0, width // 8, 1, unroll=4)
    def tok8(j8):
        ld_base = jnp.full((LANES,), j8 * 8, jnp.int32)
        row_splat = jnp.full((LANES,), j8, jnp.int32)
        for jj in range(8):
            row = plsc.load_gather(in_v, [iota, ld_base + jj])
            plsc.store_scatter(stage_v, [row_splat, st_idx[jj]], row)


def _transpose_chunk(wt_hbm, scr_hbm, in_v, stage_v, sem, lane0, width):
    lane0 = pl.multiple_of(lane0, 128)
    pltpu.sync_copy(wt_hbm.at[:, pl.ds(lane0, width)], in_v.at[:, pl.ds(0, width)])
    _transpose_loop(in_v, stage_v, width)
    pltpu.make_async_copy(
        stage_v.at[pl.ds(0, width // 8)],
        scr_hbm.at[pl.ds(pl.multiple_of(lane0 // 8, 16), width // 8)],
        sem,
    ).start()
    pltpu.make_async_copy(
        stage_v.at[pl.ds(0, width // 8)],
        scr_hbm.at[pl.ds(pl.multiple_of(lane0 // 8, 16), width // 8)],
        sem,
    ).wait()


def _tbody(wt_hbm, tail_hbm, scr_hbm, in_v, stage_v, tail_v, gsem, osem):
    nc = 2
    wid = lax.axis_index("s") * nc + lax.axis_index("c")
    base = wid * (BKT_PER_W * 128)

    def fire_in(c):
        lane0 = pl.multiple_of(base + c * WCH, 128)
        pltpu.make_async_copy(
            wt_hbm.at[:, pl.ds(lane0, WCH)],
            in_v.at[c % 2, :, pl.ds(0, WCH)], gsem).start()

    def fire_out(c):
        lane0 = pl.multiple_of(base + c * WCH, 128)
        pltpu.make_async_copy(
            stage_v.at[c % 2],
            scr_hbm.at[pl.ds(pl.multiple_of(lane0 // 8, 16), WCH // 8)],
            osem,
        ).start()

    def wait_in():
        pltpu.make_async_copy(
            wt_hbm.at[:, pl.ds(0, WCH)],
            in_v.at[0, :, pl.ds(0, WCH)], gsem).wait()

    def wait_out():
        pltpu.make_async_copy(
            stage_v.at[0], scr_hbm.at[pl.ds(0, WCH // 8)], osem).wait()

    fire_in(0)

    def chunk(c, carry):
        @pl.when(c + 1 < FULL_CH)
        def _():
            fire_in(c + 1)

        wait_in()

        @pl.when(c >= 2)
        def _():
            wait_out()

        _transpose_loop(in_v.at[c % 2], stage_v.at[c % 2], WCH)
        fire_out(c)
        return carry

    lax.fori_loop(0, FULL_CH, chunk, 0)
    wait_out()
    wait_out()
    _transpose_chunk(wt_hbm, scr_hbm, in_v.at[0], stage_v.at[0], gsem,
                     base + FULL_CH * WCH, REM)

    @pl.when(wid == 0)
    def _extra():
        _transpose_chunk(wt_hbm, scr_hbm, in_v.at[0], stage_v.at[0], gsem,
                         NW * BKT_PER_W * 128, EXTRA0 * 128)
        pltpu.sync_copy(tail_hbm, tail_v)
        iota = lax.iota(jnp.int32, LANES)

        def tok(j, carry):
            row = tail_v[j]
            plsc.store_scatter(
                stage_v.at[0],
                [jnp.full((LANES,), j // 8, jnp.int32), (j % 8) * 16 + iota],
                row,
            )
            return carry

        lax.fori_loop(0, TAIL, tok, 0, unroll=8)
        pltpu.sync_copy(
            stage_v.at[0, pl.ds(0, TAIL * 16 // 128)],
            scr_hbm.at[pl.ds(NBKT * 128 * 16 // 128, TAIL * 16 // 128)],
        )


def _gbody(tok_hbm, word_hbm, pos_hbm, out_hbm, idx_v, rows_v, stg_v, pos_v, sem):
    nc = 2
    wid = lax.axis_index("s") * nc + lax.axis_index("c")
    b0 = wid * SEQ_PER_W

    pltpu.sync_copy(tok_hbm.at[pl.ds(b0, SEQ_PER_W)], idx_v)
    pltpu.sync_copy(pos_hbm, pos_v)

    c = jnp.full((EMB,), SCALE, dtype=jnp.float32)

    def scale_pos(l, carry):
        pos_v[l] = pos_v[l] * c
        return carry

    lax.fori_loop(0, L, scale_pos, 0, unroll=4)

    iota16 = lax.iota(jnp.int32, LANES)

    for h, (l0, hh) in enumerate(((0, H0), (H0, H1))):
        def fire(s, carry):
            pltpu.make_async_copy(
                word_hbm.at[idx_v.at[s, pl.ds(l0, hh)]],
                rows_v.at[s, pl.ds(0, hh)], sem).start()
            return carry

        lax.fori_loop(0, SEQ_PER_W, fire, 0)

        def drain(s, carry):
            pltpu.make_async_copy(
                word_hbm.at[idx_v.at[0, pl.ds(l0, hh)]],
                rows_v.at[0, pl.ds(0, hh)], sem).wait()
            return carry

        lax.fori_loop(0, SEQ_PER_W, drain, 0)

        def seq(s, carry):
            @plsc.parallel_loop(0, hh, 1, unroll=4)
            def tok(i):
                val = rows_v[s, i] * c + pos_v[l0 + i]
                plsc.store_scatter(
                    stg_v,
                    [jnp.full((LANES,), i, jnp.int32), iota16,
                     jnp.full((LANES,), s, jnp.int32)],
                    val)

            return carry

        lax.fori_loop(0, SEQ_PER_W, seq, 0)

        pltpu.sync_copy(
            stg_v.at[pl.ds(0, hh), :, pl.ds(0, SEQ_PER_W)],
            out_hbm.at[pl.ds(l0, hh), :, pl.ds(b0, SEQ_PER_W)])


@jax.jit
def _embed(tok_ids, word_table, pos_table):
    mesh = plsc.VectorSubcoreMesh(core_axis_name="c", subcore_axis_name="s")

    wt_t = word_table.T
    wt_tail = lax.slice(word_table, (NBKT * 128, 0), (VOC, EMB))

    trans = pl.kernel(
        _tbody,
        out_type=jax.ShapeDtypeStruct((VOC * EMB // 128, 128), jnp.float32),
        mesh=mesh,
        scratch_types=[
            pltpu.VMEM((2, LANES, WCH + 1), jnp.float32),
            pltpu.VMEM((2, WCH // 8, 128), jnp.float32),
            pltpu.VMEM((TAIL, EMB), jnp.float32),
            pltpu.SemaphoreType.DMA,
            pltpu.SemaphoreType.DMA,
        ],
        compiler_params=pltpu.CompilerParams(
            use_tc_tiling_on_sc=True, needs_layout_passes=False),
    )
    scratch = trans(wt_t, wt_tail)
    wt_lin = scratch.reshape(VOC, EMB)

    gather = pl.kernel(
        _gbody,
        out_type=jax.ShapeDtypeStruct((L, EMB, B), jnp.float32),
        mesh=mesh,
        scratch_types=[
            pltpu.VMEM((SEQ_PER_W, L), jnp.int32),
            pltpu.VMEM((SEQ_PER_W, H0, EMB), jnp.float32),
            pltpu.VMEM((H0, EMB, 33), jnp.float32),
            pltpu.VMEM((L, EMB), jnp.float32),
            pltpu.SemaphoreType.DMA,
        ],
        compiler_params=pltpu.CompilerParams(
            use_tc_tiling_on_sc=False, needs_layout_passes=False),
    )
    out_t = gather(tok_ids, wt_lin, pos_table)
    return jnp.transpose(out_t, (2, 0, 1))


def kernel(tok_ids, word_table, pos_table):
    return _embed(tok_ids, word_table, pos_table)
